# baseline (speedup 1.0000x reference)
import functools

import jax
import jax.numpy as jnp
from jax import lax
from jax.experimental import pallas as pl
from jax.experimental.pallas import tpu as pltpu
from jax.experimental.pallas import tpu_sc as plsc

_N = 10000
_D = 128
_E = 320000
_G = 128

_NC = 2
_NS = 16
_K = 112
_GB = 3
_NG0 = 43
_NG1 = 17
_NGT = _NS * (_NG0 + _NG1)
_EPG = _GB * _K
_EP = _NGT * _EPG
_RT = 624
_TAIL = _N - _NS * _RT


def _sc_body(x_hbm, eidx_hbm, eval_hbm, out_hbm,
             acc_sh, ebuf, vbuf, rows0, rows1, rows2,
             esem, gsem0, gsem1, gsem2, ssem0, ssem1, ssem2):
    c = lax.axis_index("c")
    s = lax.axis_index("s")
    ng = jnp.where(c == 0, _NG0, _NG1)
    base = jnp.where(c == 0, s * _NG0, _NS * _NG0 + s * _NG1)
    rows = (rows0, rows1, rows2)
    gsem = (gsem0, gsem1, gsem2)
    ssem = (ssem0, ssem1, ssem2)

    zeros16 = jnp.zeros((16,), jnp.float32)

    def _zero_row(i, carry):
        for d16 in range(_D // 16):
            rows0[i, pl.ds(d16 * 16, 16)] = zeros16
        return carry

    lax.fori_loop(0, _K, _zero_row, 0)
    ofs = pl.multiple_of(s * _RT, 8)
    for q in range(_RT // _K):
        pltpu.sync_copy(rows0, acc_sh.at[pl.ds(ofs + q * _K, _K)])
    rem = _RT - (_RT // _K) * _K
    pltpu.sync_copy(rows0.at[pl.ds(0, rem)],
                    acc_sh.at[pl.ds(ofs + (_RT // _K) * _K, rem)])

    @pl.when(s == _NS - 1)
    def _():
        pltpu.sync_copy(rows0.at[pl.ds(0, _TAIL)],
                        acc_sh.at[pl.ds(_NS * _RT, _TAIL)])

    plsc.subcore_barrier()

    pltpu.sync_copy(eidx_hbm.at[base], ebuf.at[0])
    pltpu.sync_copy(eval_hbm.at[base], vbuf.at[0])
    pltpu.async_copy(x_hbm.at[ebuf.at[0, 0]], rows0, gsem0)

    def _group(t, carry):
        gi = lax.rem(t, 3)
        gin = lax.rem(t + 1, 3)

        @pl.when(t + 1 < ng)
        def _():
            pltpu.async_copy(eidx_hbm.at[base + t + 1], ebuf.at[gin], esem)
            pltpu.async_copy(eval_hbm.at[base + t + 1], vbuf.at[gin], esem)

        for j in range(_GB):
            jn = (j + 1) % 3

            pltpu.make_async_copy(
                x_hbm.at[ebuf.at[gi, 2 * j]], rows[j], gsem[j]).wait()

            if j < _GB - 1:
                @pl.when(3 * t + j >= 2)
                def _():
                    pltpu.make_async_copy(
                        rows[jn], acc_sh.at[ebuf.at[gi, 1]], ssem[jn]).wait()
                pltpu.async_copy(
                    x_hbm.at[ebuf.at[gi, 2 * (j + 1)]], rows[jn], gsem[jn])
            else:
                @pl.when(t + 1 < ng)
                def _():
                    pltpu.make_async_copy(
                        eidx_hbm.at[base + t + 1], ebuf.at[gin], esem).wait()
                    pltpu.make_async_copy(
                        eval_hbm.at[base + t + 1], vbuf.at[gin], esem).wait()
                    pltpu.make_async_copy(
                        rows[jn], acc_sh.at[ebuf.at[gi, 1]], ssem[jn]).wait()
                    pltpu.async_copy(
                        x_hbm.at[ebuf.at[gin, 0]], rows[jn], gsem[jn])

            def _scale(k16, c2):
                vv = vbuf[gi, j, pl.ds(k16 * 16, 16)]
                for t16 in range(16):
                    r = k16 * 16 + t16
                    v = vv[t16]
                    for d16 in range(_D // 16):
                        rows[j][r, pl.ds(d16 * 16, 16)] = (
                            rows[j][r, pl.ds(d16 * 16, 16)] * v)
                return c2

            lax.fori_loop(0, _K // 16, _scale, 0)

            pltpu.async_copy(
                rows[j], acc_sh.at[ebuf.at[gi, 2 * j + 1]], ssem[j],
                add=True)
        return carry

    lax.fori_loop(0, ng, _group, 0)

    for j in range(_GB):
        pltpu.make_async_copy(
            rows[j], acc_sh.at[ebuf.at[0, 1]], ssem[j]).wait()
    plsc.subcore_barrier()

    pltpu.sync_copy(acc_sh.at[pl.ds(ofs, _RT)],
                    out_hbm.at[c, pl.ds(ofs, _RT)])

    @pl.when(s == _NS - 1)
    def _():
        pltpu.sync_copy(acc_sh.at[pl.ds(_NS * _RT, _TAIL)],
                        out_hbm.at[c, pl.ds(_NS * _RT, _TAIL)])


@functools.cache
def _sc_aggregate():
    return pl.kernel(
        _sc_body,
        out_type=jax.ShapeDtypeStruct((_NC, _N, _D), jnp.float32),
        mesh=plsc.VectorSubcoreMesh(core_axis_name="c", subcore_axis_name="s",
                                    num_cores=_NC, num_subcores=_NS),
        scratch_types=[
            pltpu.VMEM_SHARED((_N, _D), jnp.float32),
            pltpu.VMEM((3, 2 * _GB, _K), jnp.int32),
            pltpu.VMEM((3, _GB, _K), jnp.float32),
            pltpu.VMEM((_K, _D), jnp.float32),
            pltpu.VMEM((_K, _D), jnp.float32),
            pltpu.VMEM((_K, _D), jnp.float32),
            pltpu.SemaphoreType.DMA,
            pltpu.SemaphoreType.DMA,
            pltpu.SemaphoreType.DMA,
            pltpu.SemaphoreType.DMA,
            pltpu.SemaphoreType.DMA,
            pltpu.SemaphoreType.DMA,
            pltpu.SemaphoreType.DMA,
        ],
    )


_NB = 10
_BN = _N // _NB


def _tc_body(aggp_ref, batch_ref, wg_ref, wf_ref, bf_ref, y_ref, fp_acc):
    i = pl.program_id(0)

    @pl.when(i == 0)
    def _():
        fp_acc[...] = jnp.zeros((_G, _D), jnp.float32)

    a = aggp_ref[0] + aggp_ref[1]
    h1 = jnp.maximum(
        jnp.dot(a, wg_ref[...], preferred_element_type=jnp.float32), 0.0)
    b = batch_ref[0]
    gid = lax.broadcasted_iota(jnp.int32, (_G, _BN), 0)
    onehot_t = (gid == b).astype(jnp.float32)
    fp_acc[...] += jnp.dot(onehot_t, h1,
                           preferred_element_type=jnp.float32)

    @pl.when(i == _NB - 1)
    def _():
        y_ref[...] = jnp.dot(fp_acc[...], wf_ref[...],
                             preferred_element_type=jnp.float32) + bf_ref[0, 0]


_tc_finish = pl.pallas_call(
    _tc_body,
    grid=(_NB,),
    in_specs=[
        pl.BlockSpec((_NC, _BN, _D), lambda i: (0, i, 0)),
        pl.BlockSpec((1, 1, _BN), lambda i: (i, 0, 0)),
        pl.BlockSpec((_D, _D), lambda i: (0, 0)),
        pl.BlockSpec((_D, 1), lambda i: (0, 0)),
        pl.BlockSpec((1, 1), lambda i: (0, 0)),
    ],
    out_specs=pl.BlockSpec((_G, 1), lambda i: (0, 0)),
    out_shape=jax.ShapeDtypeStruct((_G, 1), jnp.float32),
    scratch_shapes=[pltpu.VMEM((_G, _D), jnp.float32)],
)


def kernel(node_attr, adj_index, adj_value, batch, W_gcn, W_fc, b_fc):
    pad = _EP - _E
    dst = jnp.pad(adj_index[0], (0, pad)).reshape(_NGT, _GB, 1, _K)
    src = jnp.pad(adj_index[1], (0, pad)).reshape(_NGT, _GB, 1, _K)
    eidx = jnp.concatenate([src, dst], axis=2).reshape(_NGT, 2 * _GB, _K)
    evals = jnp.pad(adj_value, (0, pad)).reshape(_NGT, _GB, _K)
    aggp = _sc_aggregate()(node_attr, eidx, evals)
    return _tc_finish(aggp, batch.reshape(_NB, 1, _BN), W_gcn, W_fc,
                      b_fc.reshape(1, 1))


# speedup vs baseline: 1.0192x; 1.0192x over previous
import functools

import jax
import jax.numpy as jnp
from jax import lax
from jax.experimental import pallas as pl
from jax.experimental.pallas import tpu as pltpu
from jax.experimental.pallas import tpu_sc as plsc

_N = 10000
_D = 128
_E = 320000
_G = 128

_NC = 2
_NS = 16
_K = 112
_GB = 3
_NG0 = 42
_NG1 = 18
_NGT = _NS * (_NG0 + _NG1)
_EPG = _GB * _K
_EP = _NGT * _EPG
_RT = 624
_TAIL = _N - _NS * _RT


def _sc_body(x_hbm, eidx_hbm, eval_hbm, out_hbm,
             acc_sh, ebuf, vbuf, rows0, rows1, rows2,
             esem, gsem0, gsem1, gsem2, ssem0, ssem1, ssem2):
    c = lax.axis_index("c")
    s = lax.axis_index("s")
    ng = jnp.where(c == 0, _NG0, _NG1)
    base = jnp.where(c == 0, s * _NG0, _NS * _NG0 + s * _NG1)
    rows = (rows0, rows1, rows2)
    gsem = (gsem0, gsem1, gsem2)
    ssem = (ssem0, ssem1, ssem2)

    pltpu.sync_copy(eidx_hbm.at[base], ebuf.at[0])
    pltpu.sync_copy(eval_hbm.at[base], vbuf.at[0])
    pltpu.async_copy(x_hbm.at[ebuf.at[0, 0]], rows0, gsem0)

    zeros16 = jnp.zeros((16,), jnp.float32)

    def _zero_row(i, carry):
        for d16 in range(_D // 16):
            rows1[i, pl.ds(d16 * 16, 16)] = zeros16
        return carry

    lax.fori_loop(0, _K, _zero_row, 0)
    ofs = pl.multiple_of(s * _RT, 8)
    for q in range(_RT // _K):
        pltpu.sync_copy(rows1, acc_sh.at[pl.ds(ofs + q * _K, _K)])
    rem = _RT - (_RT // _K) * _K
    pltpu.sync_copy(rows1.at[pl.ds(0, rem)],
                    acc_sh.at[pl.ds(ofs + (_RT // _K) * _K, rem)])

    @pl.when(s == _NS - 1)
    def _():
        pltpu.sync_copy(rows1.at[pl.ds(0, _TAIL)],
                        acc_sh.at[pl.ds(_NS * _RT, _TAIL)])

    plsc.subcore_barrier()

    def _group(t, carry):
        gi = lax.rem(t, 3)
        gin = lax.rem(t + 1, 3)

        @pl.when(t + 1 < ng)
        def _():
            pltpu.async_copy(eidx_hbm.at[base + t + 1], ebuf.at[gin], esem)
            pltpu.async_copy(eval_hbm.at[base + t + 1], vbuf.at[gin], esem)

        for j in range(_GB):
            jn = (j + 1) % 3

            pltpu.make_async_copy(
                x_hbm.at[ebuf.at[gi, 2 * j]], rows[j], gsem[j]).wait()

            if j < _GB - 1:
                @pl.when(3 * t + j >= 2)
                def _():
                    pltpu.make_async_copy(
                        rows[jn], acc_sh.at[ebuf.at[gi, 1]], ssem[jn]).wait()
                pltpu.async_copy(
                    x_hbm.at[ebuf.at[gi, 2 * (j + 1)]], rows[jn], gsem[jn])
            else:
                @pl.when(t + 1 < ng)
                def _():
                    pltpu.make_async_copy(
                        eidx_hbm.at[base + t + 1], ebuf.at[gin], esem).wait()
                    pltpu.make_async_copy(
                        eval_hbm.at[base + t + 1], vbuf.at[gin], esem).wait()
                    pltpu.make_async_copy(
                        rows[jn], acc_sh.at[ebuf.at[gi, 1]], ssem[jn]).wait()
                    pltpu.async_copy(
                        x_hbm.at[ebuf.at[gin, 0]], rows[jn], gsem[jn])

            def _scale(k16, c2):
                vv = vbuf[gi, j, pl.ds(k16 * 16, 16)]
                for t16 in range(16):
                    r = k16 * 16 + t16
                    v = vv[t16]
                    for d16 in range(_D // 16):
                        rows[j][r, pl.ds(d16 * 16, 16)] = (
                            rows[j][r, pl.ds(d16 * 16, 16)] * v)
                return c2

            lax.fori_loop(0, _K // 16, _scale, 0)

            pltpu.async_copy(
                rows[j], acc_sh.at[ebuf.at[gi, 2 * j + 1]], ssem[j],
                add=True)
        return carry

    lax.fori_loop(0, ng, _group, 0)

    for j in range(_GB):
        pltpu.make_async_copy(
            rows[j], acc_sh.at[ebuf.at[0, 1]], ssem[j]).wait()
    plsc.subcore_barrier()

    pltpu.sync_copy(acc_sh.at[pl.ds(ofs, _RT)],
                    out_hbm.at[c, pl.ds(ofs, _RT)])

    @pl.when(s == _NS - 1)
    def _():
        pltpu.sync_copy(acc_sh.at[pl.ds(_NS * _RT, _TAIL)],
                        out_hbm.at[c, pl.ds(_NS * _RT, _TAIL)])


@functools.cache
def _sc_aggregate():
    return pl.kernel(
        _sc_body,
        out_type=jax.ShapeDtypeStruct((_NC, _N, _D), jnp.float32),
        mesh=plsc.VectorSubcoreMesh(core_axis_name="c", subcore_axis_name="s",
                                    num_cores=_NC, num_subcores=_NS),
        scratch_types=[
            pltpu.VMEM_SHARED((_N, _D), jnp.float32),
            pltpu.VMEM((3, 2 * _GB, _K), jnp.int32),
            pltpu.VMEM((3, _GB, _K), jnp.float32),
            pltpu.VMEM((_K, _D), jnp.float32),
            pltpu.VMEM((_K, _D), jnp.float32),
            pltpu.VMEM((_K, _D), jnp.float32),
            pltpu.SemaphoreType.DMA,
            pltpu.SemaphoreType.DMA,
            pltpu.SemaphoreType.DMA,
            pltpu.SemaphoreType.DMA,
            pltpu.SemaphoreType.DMA,
            pltpu.SemaphoreType.DMA,
            pltpu.SemaphoreType.DMA,
        ],
    )


_NB = 10
_BN = _N // _NB


def _tc_body(aggp_ref, batch_ref, wg_ref, wf_ref, bf_ref, y_ref, fp_acc):
    i = pl.program_id(0)

    @pl.when(i == 0)
    def _():
        fp_acc[...] = jnp.zeros((_G, _D), jnp.float32)

    a = aggp_ref[0] + aggp_ref[1]
    h1 = jnp.maximum(
        jnp.dot(a, wg_ref[...], preferred_element_type=jnp.float32), 0.0)
    b = batch_ref[0]
    gid = lax.broadcasted_iota(jnp.int32, (_G, _BN), 0)
    onehot_t = (gid == b).astype(jnp.float32)
    fp_acc[...] += jnp.dot(onehot_t, h1,
                           preferred_element_type=jnp.float32)

    @pl.when(i == _NB - 1)
    def _():
        y_ref[...] = jnp.dot(fp_acc[...], wf_ref[...],
                             preferred_element_type=jnp.float32) + bf_ref[0, 0]


_tc_finish = pl.pallas_call(
    _tc_body,
    grid=(_NB,),
    in_specs=[
        pl.BlockSpec((_NC, _BN, _D), lambda i: (0, i, 0)),
        pl.BlockSpec((1, 1, _BN), lambda i: (i, 0, 0)),
        pl.BlockSpec((_D, _D), lambda i: (0, 0)),
        pl.BlockSpec((_D, 1), lambda i: (0, 0)),
        pl.BlockSpec((1, 1), lambda i: (0, 0)),
    ],
    out_specs=pl.BlockSpec((_G, 1), lambda i: (0, 0)),
    out_shape=jax.ShapeDtypeStruct((_G, 1), jnp.float32),
    scratch_shapes=[pltpu.VMEM((_G, _D), jnp.float32)],
)


def kernel(node_attr, adj_index, adj_value, batch, W_gcn, W_fc, b_fc):
    pad = _EP - _E
    dst = jnp.pad(adj_index[0], (0, pad)).reshape(_NGT, _GB, 1, _K)
    src = jnp.pad(adj_index[1], (0, pad)).reshape(_NGT, _GB, 1, _K)
    eidx = jnp.concatenate([src, dst], axis=2).reshape(_NGT, 2 * _GB, _K)
    evals = jnp.pad(adj_value, (0, pad)).reshape(_NGT, _GB, _K)
    aggp = _sc_aggregate()(node_attr, eidx, evals)
    return _tc_finish(aggp, batch.reshape(_NB, 1, _BN), W_gcn, W_fc,
                      b_fc.reshape(1, 1))
